# Initial kernel scaffold; baseline (speedup 1.0000x reference)
import jax
import jax.numpy as jnp
from jax.experimental import pallas as pl


def kernel(x, edge_index, gc1_w1, gc1_b1, gc1_w2, gc1_b2, gc2_w1, gc2_b1, gc2_w2, gc2_b2, gc3_w1, gc3_b1, gc3_w2, gc3_b2, lin1_w, lin1_b, lin2_w, lin2_b):
    raise NotImplementedError("write your pallas kernel here")


# speedup vs baseline: 6.4175x; 6.4175x over previous
import functools

import jax
import jax.numpy as jnp
from jax import lax
from jax.experimental import pallas as pl
from jax.experimental.pallas import tpu as pltpu
from jax.experimental.pallas import tpu_sc as plsc

N = 10000
D = 128
E = 320000
NCLASS = 40
NC = 2
NS = 16
NW = NC * NS
EPW = E // NW
CHUNK = 80
NCHUNK = EPW // CHUNK
NP = 10240
RPT = NP // NS

BM = 1000
GRID = N // BM


def _seg_sum_body(x_hbm, src_hbm, dst_hbm, zero_hbm, out_hbm,
                  src_v, dst_v, rows_v, agg_sh, sem):
    c = lax.axis_index("c")
    s = lax.axis_index("s")
    wid = c * NS + s

    pltpu.sync_copy(zero_hbm.at[pl.ds(s * RPT, RPT)],
                    agg_sh.at[pl.ds(s * RPT, RPT)])
    pltpu.sync_copy(src_hbm.at[wid], src_v)
    pltpu.sync_copy(dst_hbm.at[wid], dst_v)
    plsc.subcore_barrier()

    def body(i, carry):
        pltpu.async_copy(x_hbm.at[src_v.at[i]], rows_v, sem).wait()
        pltpu.sync_copy(rows_v, agg_sh.at[dst_v.at[i]], add=True)
        return carry

    lax.fori_loop(0, NCHUNK, body, 0)

    plsc.subcore_barrier()
    pltpu.sync_copy(agg_sh.at[pl.ds(s * RPT, RPT)],
                    out_hbm.at[c, pl.ds(s * RPT, RPT)])


_seg_sum = pl.kernel(
    _seg_sum_body,
    out_type=jax.ShapeDtypeStruct((NC, NP, D), jnp.float32),
    mesh=plsc.VectorSubcoreMesh(core_axis_name="c", subcore_axis_name="s"),
    scratch_types=[
        pltpu.VMEM((NCHUNK, CHUNK), jnp.int32),
        pltpu.VMEM((NCHUNK, CHUNK), jnp.int32),
        pltpu.VMEM((CHUNK, D), jnp.float32),
        pltpu.VMEM_SHARED((NP, D), jnp.float32),
        pltpu.SemaphoreType.DMA,
    ],
)


def _mlp_body(x_ref, p0_ref, p1_ref, w1_ref, b1_ref, w2_ref, b2_ref, o_ref):
    h = x_ref[...] + p0_ref[...] + p1_ref[...]
    h = jnp.maximum(
        jnp.dot(h, w1_ref[...], preferred_element_type=jnp.float32)
        + b1_ref[...], 0.0)
    h = jnp.maximum(
        jnp.dot(h, w2_ref[...], preferred_element_type=jnp.float32)
        + b2_ref[...], 0.0)
    o_ref[...] = h


_row_spec = pl.BlockSpec((BM, D), lambda i: (i, 0))


def _full(shape):
    return pl.BlockSpec(shape, lambda i: (0,) * len(shape))


_mlp = pl.pallas_call(
    _mlp_body,
    grid=(GRID,),
    in_specs=[_row_spec, _row_spec, _row_spec,
              _full((D, D)), _full((1, D)), _full((D, D)), _full((1, D))],
    out_specs=_row_spec,
    out_shape=jax.ShapeDtypeStruct((N, D), jnp.float32),
)


def _head_body(h1_ref, h2_ref, h3_ref, wa_ref, wb_ref, wc_ref, b1_ref,
               w2_ref, b2_ref, o_ref):
    t = (jnp.dot(h1_ref[...], wa_ref[...], preferred_element_type=jnp.float32)
         + jnp.dot(h2_ref[...], wb_ref[...], preferred_element_type=jnp.float32)
         + jnp.dot(h3_ref[...], wc_ref[...], preferred_element_type=jnp.float32)
         + b1_ref[...])
    t = jnp.maximum(t, 0.0)
    z = (jnp.dot(t, w2_ref[...], preferred_element_type=jnp.float32)
         + b2_ref[...])
    valid = lax.broadcasted_iota(jnp.int32, z.shape, 1) < NCLASS
    zm = jnp.where(valid, z, -jnp.inf)
    m = jnp.max(zm, axis=1, keepdims=True)
    ls = jnp.log(jnp.sum(jnp.exp(zm - m), axis=1, keepdims=True)) + m
    o_ref[...] = z - ls


_head = pl.pallas_call(
    _head_body,
    grid=(GRID,),
    in_specs=[_row_spec, _row_spec, _row_spec,
              _full((D, 3 * D)), _full((D, 3 * D)), _full((D, 3 * D)),
              _full((1, 3 * D)), _full((3 * D, D)), _full((1, D))],
    out_specs=_row_spec,
    out_shape=jax.ShapeDtypeStruct((N, D), jnp.float32),
)


def kernel(x, edge_index, gc1_w1, gc1_b1, gc1_w2, gc1_b2, gc2_w1, gc2_b1,
           gc2_w2, gc2_b2, gc3_w1, gc3_b1, gc3_w2, gc3_b2, lin1_w, lin1_b,
           lin2_w, lin2_b):
    src = edge_index[0].astype(jnp.int32).reshape(NW, NCHUNK, CHUNK)
    dst = edge_index[1].astype(jnp.int32).reshape(NW, NCHUNK, CHUNK)
    zero = jnp.zeros((NP, D), jnp.float32)

    h = x
    hs = []
    for w1, b1, w2, b2 in ((gc1_w1, gc1_b1, gc1_w2, gc1_b2),
                           (gc2_w1, gc2_b1, gc2_w2, gc2_b2),
                           (gc3_w1, gc3_b1, gc3_w2, gc3_b2)):
        parts = _seg_sum(h, src, dst, zero)
        h = _mlp(h, parts[0, :N], parts[1, :N], w1, b1.reshape(1, D),
                 w2, b2.reshape(1, D))
        hs.append(h)

    w2p = jnp.zeros((3 * D, D), jnp.float32).at[:, :NCLASS].set(lin2_w)
    b2p = jnp.zeros((1, D), jnp.float32).at[0, :NCLASS].set(lin2_b)
    out = _head(hs[0], hs[1], hs[2],
                lin1_w[:D], lin1_w[D:2 * D], lin1_w[2 * D:],
                lin1_b.reshape(1, 3 * D), w2p, b2p)
    return out[:, :NCLASS]
